# baseline (speedup 1.0000x reference)
import functools

import jax
import jax.numpy as jnp
from jax import lax
from jax.experimental import pallas as pl
from jax.experimental.pallas import tpu as pltpu
from jax.experimental.pallas import tpu_sc as plsc

D = 128
NC = 2
NS = 16
NW = NC * NS
N_PAD = 10240
ROWS_PER_SUB = N_PAD // NS
CHUNK = 128
GRP = 8
GW = 10
NGRP_TOT = NW * GW
E_PAD = NGRP_TOT * GRP * CHUNK
TC_BLK = 1024

_sc_mesh = functools.partial(
    plsc.VectorSubcoreMesh, core_axis_name="c", subcore_axis_name="s"
)


def _sc_degree_hist(dst_rows):
    @functools.partial(
        pl.kernel,
        out_type=jax.ShapeDtypeStruct((NC, N_PAD), jnp.float32),
        mesh=_sc_mesh(),
        scratch_types=[
            pltpu.VMEM_SHARED((N_PAD,), jnp.float32),
            pltpu.VMEM((GW, GRP, CHUNK), jnp.int32),
            pltpu.VMEM((ROWS_PER_SUB,), jnp.float32),
            pltpu.VMEM((CHUNK,), jnp.float32),
        ],
    )
    def hist(dst_hbm, out_hbm, acc, dst_v, zero_v, ones_v):
        cid = lax.axis_index("c")
        sid = lax.axis_index("s")

        @pl.loop(0, ROWS_PER_SUB, step=16)
        def _(i):
            zero_v[pl.ds(i, 16)] = jnp.zeros((16,), jnp.float32)

        @pl.loop(0, CHUNK, step=16)
        def _(i):
            ones_v[pl.ds(i, 16)] = jnp.ones((16,), jnp.float32)

        pltpu.sync_copy(zero_v, acc.at[pl.ds(sid * ROWS_PER_SUB, ROWS_PER_SUB)])
        plsc.subcore_barrier()

        wid = sid * NC + cid
        pltpu.sync_copy(dst_hbm.at[pl.ds(wid * GW, GW)], dst_v)

        @pl.loop(0, GW)
        def _(t):
            @pl.loop(0, GRP)
            def _(k):
                pltpu.sync_copy(ones_v, acc.at[dst_v.at[t, k]], add=True)

        plsc.subcore_barrier()
        pltpu.sync_copy(
            acc.at[pl.ds(sid * ROWS_PER_SUB, ROWS_PER_SUB)],
            out_hbm.at[cid, pl.ds(sid * ROWS_PER_SUB, ROWS_PER_SUB)],
        )

    return hist(dst_rows)


def _sc_hop(g, src_rows, dst_rows):
    @functools.partial(
        pl.kernel,
        out_type=jax.ShapeDtypeStruct((NC, N_PAD, D), jnp.float32),
        mesh=_sc_mesh(),
        scratch_types=[
            pltpu.VMEM_SHARED((N_PAD, D), jnp.float32),
            pltpu.VMEM((2, GRP, CHUNK), jnp.int32),
            pltpu.VMEM((2, GRP, CHUNK), jnp.int32),
            pltpu.VMEM((CHUNK, D), jnp.float32),
            pltpu.VMEM((CHUNK, D), jnp.float32),
            pltpu.SemaphoreType.DMA,
            pltpu.SemaphoreType.DMA,
            pltpu.SemaphoreType.DMA,
            pltpu.SemaphoreType.DMA,
            pltpu.SemaphoreType.DMA,
        ],
    )
    def hop(g_hbm, src_hbm, dst_hbm, out_hbm, acc, src_v, dst_v, rows0, rows1,
            sem0, sem0b, sem1, sem1b, semi):
        cid = lax.axis_index("c")
        sid = lax.axis_index("s")

        @pl.loop(0, CHUNK)
        def _(r):
            @pl.loop(0, D, step=16)
            def _(c):
                rows0[r, pl.ds(c, 16)] = jnp.zeros((16,), jnp.float32)

        @pl.loop(0, ROWS_PER_SUB, step=CHUNK)
        def _(i):
            pltpu.sync_copy(rows0, acc.at[pl.ds(sid * ROWS_PER_SUB + i, CHUNK)])

        plsc.subcore_barrier()

        def g_start(sv, k, buf, sem):
            row = sv.at[k]
            pltpu.async_copy(g_hbm.at[row.at[pl.ds(0, 64)]],
                             buf.at[pl.ds(0, 64)], sem[0])
            pltpu.async_copy(g_hbm.at[row.at[pl.ds(64, 64)]],
                             buf.at[pl.ds(64, 64)], sem[1])

        def g_wait(sv, k, buf, sem):
            row = sv.at[k]
            pltpu.make_async_copy(g_hbm.at[row.at[pl.ds(0, 64)]],
                                  buf.at[pl.ds(0, 64)], sem[0]).wait()
            pltpu.make_async_copy(g_hbm.at[row.at[pl.ds(64, 64)]],
                                  buf.at[pl.ds(64, 64)], sem[1]).wait()

        def sc_add(dv, k, buf):
            pltpu.sync_copy(buf, acc.at[dv.at[k]], add=True)

        def run(bg, ngr):
            pltpu.sync_copy(src_hbm.at[bg], src_v.at[0])
            pltpu.sync_copy(dst_hbm.at[bg], dst_v.at[0])

            def group(t, p):
                @pl.when(t + 1 < ngr)
                def _():
                    pltpu.async_copy(src_hbm.at[bg + t + 1], src_v.at[1 - p], semi)
                    pltpu.async_copy(dst_hbm.at[bg + t + 1], dst_v.at[1 - p], semi)

                sv = src_v.at[p]
                dv = dst_v.at[p]
                g_start(sv, 0, rows0, (sem0, sem0b))

                @pl.loop(0, GRP, step=2)
                def _(k):
                    g_start(sv, k + 1, rows1, (sem1, sem1b))
                    g_wait(sv, k, rows0, (sem0, sem0b))
                    sc_add(dv, k, rows0)

                    @pl.when(k + 2 < GRP)
                    def _():
                        g_start(sv, k + 2, rows0, (sem0, sem0b))

                    g_wait(sv, k + 1, rows1, (sem1, sem1b))
                    sc_add(dv, k + 1, rows1)

                @pl.when(t + 1 < ngr)
                def _():
                    pltpu.make_async_copy(src_hbm.at[bg + t + 1],
                                          src_v.at[1 - p], semi).wait()
                    pltpu.make_async_copy(dst_hbm.at[bg + t + 1],
                                          dst_v.at[1 - p], semi).wait()

            @pl.loop(0, ngr, step=2)
            def _(t):
                group(t, 0)
                group(t + 1, 1)

        run((sid * NC + cid) * GW, GW)

        plsc.subcore_barrier()
        pltpu.sync_copy(
            acc.at[pl.ds(sid * ROWS_PER_SUB, ROWS_PER_SUB)],
            out_hbm.at[cid, pl.ds(sid * ROWS_PER_SUB, ROWS_PER_SUB)],
        )

    return hop(g, src_rows, dst_rows)


def _deg_col(deg_ref):
    return jnp.sum(deg_ref[...], axis=1, keepdims=True) + 1.0


def _tc_lin_scale(x_pad, W, degt):
    def body(x_ref, w_ref, deg_ref, g_ref):
        y = lax.dot_general(x_ref[...], w_ref[...], (((1,), (1,)), ((), ())),
                            preferred_element_type=jnp.float32)
        g_ref[...] = y * lax.rsqrt(_deg_col(deg_ref))

    return pl.pallas_call(
        body,
        grid=(N_PAD // TC_BLK,),
        in_specs=[
            pl.BlockSpec((TC_BLK, D), lambda i: (i, 0)),
            pl.BlockSpec((D, D), lambda i: (0, 0)),
            pl.BlockSpec((TC_BLK, NC), lambda i: (i, 0)),
        ],
        out_specs=pl.BlockSpec((TC_BLK, D), lambda i: (i, 0)),
        out_shape=jax.ShapeDtypeStruct((N_PAD, D), jnp.float32),
    )(x_pad, W, degt)


def _tc_mid(accp, g1, degt):
    def body(a_ref, g_ref, deg_ref, o_ref):
        s = a_ref[0] + a_ref[1] + g_ref[...]
        o_ref[...] = s / _deg_col(deg_ref)

    return pl.pallas_call(
        body,
        grid=(N_PAD // TC_BLK,),
        in_specs=[
            pl.BlockSpec((NC, TC_BLK, D), lambda i: (0, i, 0)),
            pl.BlockSpec((TC_BLK, D), lambda i: (i, 0)),
            pl.BlockSpec((TC_BLK, NC), lambda i: (i, 0)),
        ],
        out_specs=pl.BlockSpec((TC_BLK, D), lambda i: (i, 0)),
        out_shape=jax.ShapeDtypeStruct((N_PAD, D), jnp.float32),
    )(accp, g1, degt)


def _tc_final(accp, g2, degt, b2d):
    def body(a_ref, g_ref, deg_ref, b_ref, o_ref):
        s = a_ref[0] + a_ref[1] + g_ref[...]
        o_ref[...] = s * lax.rsqrt(_deg_col(deg_ref)) + b_ref[...]

    return pl.pallas_call(
        body,
        grid=(N_PAD // TC_BLK,),
        in_specs=[
            pl.BlockSpec((NC, TC_BLK, D), lambda i: (0, i, 0)),
            pl.BlockSpec((TC_BLK, D), lambda i: (i, 0)),
            pl.BlockSpec((TC_BLK, NC), lambda i: (i, 0)),
            pl.BlockSpec((1, D), lambda i: (0, 0)),
        ],
        out_specs=pl.BlockSpec((TC_BLK, D), lambda i: (i, 0)),
        out_shape=jax.ShapeDtypeStruct((N_PAD, D), jnp.float32),
    )(accp, g2, degt, b2d)


def kernel(x, edge_index, W, b):
    n = x.shape[0]
    e = edge_index.shape[1]
    src = edge_index[0].astype(jnp.int32)
    dst = edge_index[1].astype(jnp.int32)
    pad = n + (jnp.arange(E_PAD - e, dtype=jnp.int32) % (N_PAD - n))
    src_rows = jnp.concatenate([src, pad]).reshape(NGRP_TOT, GRP, CHUNK)
    dst_rows = jnp.concatenate([dst, pad]).reshape(NGRP_TOT, GRP, CHUNK)
    x_pad = jnp.pad(x, ((0, N_PAD - n), (0, 0)))

    degp = _sc_degree_hist(dst_rows)
    degt = degp.T
    g1 = _tc_lin_scale(x_pad, W, degt)
    acc1 = _sc_hop(g1, src_rows, dst_rows)
    g2 = _tc_mid(acc1, g1, degt)
    acc2 = _sc_hop(g2, src_rows, dst_rows)
    out = _tc_final(acc2, g2, degt, b.reshape(1, D))
    return out[:n]


# speedup vs baseline: 1.0818x; 1.0818x over previous
import functools

import jax
import jax.numpy as jnp
from jax import lax
from jax.experimental import pallas as pl
from jax.experimental.pallas import tpu as pltpu
from jax.experimental.pallas import tpu_sc as plsc

D = 128
NC = 2
NS = 16
NW = NC * NS
N_PAD = 10240
ROWS_PER_SUB = N_PAD // NS
CHUNK = 128
GRP = 8
GW = 10
NGRP_TOT = NW * GW
E_PAD = NGRP_TOT * GRP * CHUNK
TC_BLK = 1024

_sc_mesh = functools.partial(
    plsc.VectorSubcoreMesh, core_axis_name="c", subcore_axis_name="s"
)


def _sc_degree_hist(dst_rows):
    @functools.partial(
        pl.kernel,
        out_type=jax.ShapeDtypeStruct((NC, N_PAD), jnp.float32),
        mesh=_sc_mesh(),
        scratch_types=[
            pltpu.VMEM_SHARED((N_PAD,), jnp.float32),
            pltpu.VMEM((GW, GRP, CHUNK), jnp.int32),
            pltpu.VMEM((ROWS_PER_SUB,), jnp.float32),
            pltpu.VMEM((CHUNK,), jnp.float32),
        ],
    )
    def hist(dst_hbm, out_hbm, acc, dst_v, zero_v, ones_v):
        cid = lax.axis_index("c")
        sid = lax.axis_index("s")

        @pl.loop(0, ROWS_PER_SUB, step=16)
        def _(i):
            zero_v[pl.ds(i, 16)] = jnp.zeros((16,), jnp.float32)

        @pl.loop(0, CHUNK, step=16)
        def _(i):
            ones_v[pl.ds(i, 16)] = jnp.ones((16,), jnp.float32)

        pltpu.sync_copy(zero_v, acc.at[pl.ds(sid * ROWS_PER_SUB, ROWS_PER_SUB)])
        plsc.subcore_barrier()

        wid = sid * NC + cid
        pltpu.sync_copy(dst_hbm.at[pl.ds(wid * GW, GW)], dst_v)

        @pl.loop(0, GW)
        def _(t):
            @pl.loop(0, GRP)
            def _(k):
                pltpu.sync_copy(ones_v, acc.at[dst_v.at[t, k]], add=True)

        plsc.subcore_barrier()
        pltpu.sync_copy(
            acc.at[pl.ds(sid * ROWS_PER_SUB, ROWS_PER_SUB)],
            out_hbm.at[cid, pl.ds(sid * ROWS_PER_SUB, ROWS_PER_SUB)],
        )

    return hist(dst_rows)


def _sc_hop(g, src_rows, dst_rows):
    @functools.partial(
        pl.kernel,
        out_type=jax.ShapeDtypeStruct((NC, N_PAD, D), jnp.float32),
        mesh=_sc_mesh(),
        scratch_types=[
            pltpu.VMEM_SHARED((N_PAD, D), jnp.float32),
            pltpu.VMEM((2, GRP, CHUNK), jnp.int32),
            pltpu.VMEM((2, GRP, CHUNK), jnp.int32),
            pltpu.VMEM((CHUNK, D), jnp.float32),
            pltpu.VMEM((CHUNK, D), jnp.float32),
            pltpu.SemaphoreType.DMA,
            pltpu.SemaphoreType.DMA,
            pltpu.SemaphoreType.DMA,
        ],
    )
    def hop(g_hbm, src_hbm, dst_hbm, out_hbm, acc, src_v, dst_v, rows0, rows1,
            sem0, sem1, semi):
        cid = lax.axis_index("c")
        sid = lax.axis_index("s")

        @pl.loop(0, CHUNK)
        def _(r):
            @pl.loop(0, D, step=16)
            def _(c):
                rows0[r, pl.ds(c, 16)] = jnp.zeros((16,), jnp.float32)

        @pl.loop(0, ROWS_PER_SUB, step=CHUNK)
        def _(i):
            pltpu.sync_copy(rows0, acc.at[pl.ds(sid * ROWS_PER_SUB + i, CHUNK)])

        plsc.subcore_barrier()

        def g_start(sv, k, buf, sem):
            pltpu.async_copy(g_hbm.at[sv.at[k]], buf, sem[0])

        def g_wait(sv, k, buf, sem):
            pltpu.make_async_copy(g_hbm.at[sv.at[k]], buf, sem[0]).wait()

        def sc_add(dv, k, buf):
            pltpu.sync_copy(buf, acc.at[dv.at[k]], add=True)

        def run(bg, ngr):
            pltpu.sync_copy(src_hbm.at[bg], src_v.at[0])
            pltpu.sync_copy(dst_hbm.at[bg], dst_v.at[0])
            g_start(src_v.at[0], 0, rows0, (sem0,))

            def group(t, p):
                @pl.when(t + 1 < ngr)
                def _():
                    pltpu.async_copy(src_hbm.at[bg + t + 1], src_v.at[1 - p], semi)
                    pltpu.async_copy(dst_hbm.at[bg + t + 1], dst_v.at[1 - p], semi)

                sv = src_v.at[p]
                dv = dst_v.at[p]
                svn = src_v.at[1 - p]

                @pl.loop(0, GRP, step=2)
                def _(k):
                    g_start(sv, k + 1, rows1, (sem1,))
                    g_wait(sv, k, rows0, (sem0,))
                    sc_add(dv, k, rows0)

                    @pl.when(k + 2 < GRP)
                    def _():
                        g_start(sv, k + 2, rows0, (sem0,))

                    @pl.when((k + 2 >= GRP) & (t + 1 < ngr))
                    def _():
                        pltpu.make_async_copy(src_hbm.at[bg + t + 1],
                                              src_v.at[1 - p], semi).wait()
                        pltpu.make_async_copy(dst_hbm.at[bg + t + 1],
                                              dst_v.at[1 - p], semi).wait()
                        g_start(svn, 0, rows0, (sem0,))

                    g_wait(sv, k + 1, rows1, (sem1,))
                    sc_add(dv, k + 1, rows1)

            @pl.loop(0, ngr, step=2)
            def _(t):
                group(t, 0)
                group(t + 1, 1)

        run((sid * NC + cid) * GW, GW)

        plsc.subcore_barrier()
        pltpu.sync_copy(
            acc.at[pl.ds(sid * ROWS_PER_SUB, ROWS_PER_SUB)],
            out_hbm.at[cid, pl.ds(sid * ROWS_PER_SUB, ROWS_PER_SUB)],
        )

    return hop(g, src_rows, dst_rows)


def _deg_col(deg_ref):
    return jnp.sum(deg_ref[...], axis=1, keepdims=True) + 1.0


def _tc_lin_scale(x_pad, W, degt):
    def body(x_ref, w_ref, deg_ref, g_ref):
        y = lax.dot_general(x_ref[...], w_ref[...], (((1,), (1,)), ((), ())),
                            preferred_element_type=jnp.float32)
        g_ref[...] = y * lax.rsqrt(_deg_col(deg_ref))

    return pl.pallas_call(
        body,
        grid=(N_PAD // TC_BLK,),
        in_specs=[
            pl.BlockSpec((TC_BLK, D), lambda i: (i, 0)),
            pl.BlockSpec((D, D), lambda i: (0, 0)),
            pl.BlockSpec((TC_BLK, NC), lambda i: (i, 0)),
        ],
        out_specs=pl.BlockSpec((TC_BLK, D), lambda i: (i, 0)),
        out_shape=jax.ShapeDtypeStruct((N_PAD, D), jnp.float32),
    )(x_pad, W, degt)


def _tc_mid(accp, g1, degt):
    def body(a_ref, g_ref, deg_ref, o_ref):
        s = a_ref[0] + a_ref[1] + g_ref[...]
        o_ref[...] = s / _deg_col(deg_ref)

    return pl.pallas_call(
        body,
        grid=(N_PAD // TC_BLK,),
        in_specs=[
            pl.BlockSpec((NC, TC_BLK, D), lambda i: (0, i, 0)),
            pl.BlockSpec((TC_BLK, D), lambda i: (i, 0)),
            pl.BlockSpec((TC_BLK, NC), lambda i: (i, 0)),
        ],
        out_specs=pl.BlockSpec((TC_BLK, D), lambda i: (i, 0)),
        out_shape=jax.ShapeDtypeStruct((N_PAD, D), jnp.float32),
    )(accp, g1, degt)


def _tc_final(accp, g2, degt, b2d):
    def body(a_ref, g_ref, deg_ref, b_ref, o_ref):
        s = a_ref[0] + a_ref[1] + g_ref[...]
        o_ref[...] = s * lax.rsqrt(_deg_col(deg_ref)) + b_ref[...]

    return pl.pallas_call(
        body,
        grid=(N_PAD // TC_BLK,),
        in_specs=[
            pl.BlockSpec((NC, TC_BLK, D), lambda i: (0, i, 0)),
            pl.BlockSpec((TC_BLK, D), lambda i: (i, 0)),
            pl.BlockSpec((TC_BLK, NC), lambda i: (i, 0)),
            pl.BlockSpec((1, D), lambda i: (0, 0)),
        ],
        out_specs=pl.BlockSpec((TC_BLK, D), lambda i: (i, 0)),
        out_shape=jax.ShapeDtypeStruct((N_PAD, D), jnp.float32),
    )(accp, g2, degt, b2d)


def kernel(x, edge_index, W, b):
    n = x.shape[0]
    e = edge_index.shape[1]
    src = edge_index[0].astype(jnp.int32)
    dst = edge_index[1].astype(jnp.int32)
    pad = n + (jnp.arange(E_PAD - e, dtype=jnp.int32) % (N_PAD - n))
    src_rows = jnp.concatenate([src, pad]).reshape(NGRP_TOT, GRP, CHUNK)
    dst_rows = jnp.concatenate([dst, pad]).reshape(NGRP_TOT, GRP, CHUNK)
    x_pad = jnp.pad(x, ((0, N_PAD - n), (0, 0)))

    degp = _sc_degree_hist(dst_rows)
    degt = degp.T
    g1 = _tc_lin_scale(x_pad, W, degt)
    acc1 = _sc_hop(g1, src_rows, dst_rows)
    g2 = _tc_mid(acc1, g1, degt)
    acc2 = _sc_hop(g2, src_rows, dst_rows)
    out = _tc_final(acc2, g2, degt, b.reshape(1, D))
    return out[:n]


# speedup vs baseline: 1.1017x; 1.0184x over previous
import functools

import jax
import jax.numpy as jnp
from jax import lax
from jax.experimental import pallas as pl
from jax.experimental.pallas import tpu as pltpu
from jax.experimental.pallas import tpu_sc as plsc

D = 128
NC = 2
NS = 16
NW = NC * NS
N_PAD = 10240
ROWS_PER_SUB = N_PAD // NS
CHUNK = 128
GRP = 8
GW = 10
NGRP_TOT = NW * GW
E_PAD = NGRP_TOT * GRP * CHUNK
TC_BLK = 2048

_sc_mesh = functools.partial(
    plsc.VectorSubcoreMesh, core_axis_name="c", subcore_axis_name="s"
)


def _sc_degree_hist(dst_rows):
    @functools.partial(
        pl.kernel,
        out_type=jax.ShapeDtypeStruct((NC, N_PAD), jnp.float32),
        mesh=_sc_mesh(),
        scratch_types=[
            pltpu.VMEM_SHARED((N_PAD,), jnp.float32),
            pltpu.VMEM((GW, GRP, CHUNK), jnp.int32),
            pltpu.VMEM((ROWS_PER_SUB,), jnp.float32),
            pltpu.VMEM((CHUNK,), jnp.float32),
        ],
    )
    def hist(dst_hbm, out_hbm, acc, dst_v, zero_v, ones_v):
        cid = lax.axis_index("c")
        sid = lax.axis_index("s")

        @pl.loop(0, ROWS_PER_SUB, step=16)
        def _(i):
            zero_v[pl.ds(i, 16)] = jnp.zeros((16,), jnp.float32)

        @pl.loop(0, CHUNK, step=16)
        def _(i):
            ones_v[pl.ds(i, 16)] = jnp.ones((16,), jnp.float32)

        pltpu.sync_copy(zero_v, acc.at[pl.ds(sid * ROWS_PER_SUB, ROWS_PER_SUB)])
        plsc.subcore_barrier()

        wid = sid * NC + cid
        pltpu.sync_copy(dst_hbm.at[pl.ds(wid * GW, GW)], dst_v)

        @pl.loop(0, GW)
        def _(t):
            @pl.loop(0, GRP)
            def _(k):
                pltpu.sync_copy(ones_v, acc.at[dst_v.at[t, k]], add=True)

        plsc.subcore_barrier()
        pltpu.sync_copy(
            acc.at[pl.ds(sid * ROWS_PER_SUB, ROWS_PER_SUB)],
            out_hbm.at[cid, pl.ds(sid * ROWS_PER_SUB, ROWS_PER_SUB)],
        )

    return hist(dst_rows)


def _sc_hop(g, src_rows, dst_rows):
    @functools.partial(
        pl.kernel,
        out_type=jax.ShapeDtypeStruct((NC, N_PAD, D), jnp.float32),
        mesh=_sc_mesh(),
        scratch_types=[
            pltpu.VMEM_SHARED((N_PAD, D), jnp.float32),
            pltpu.VMEM((2, GRP, CHUNK), jnp.int32),
            pltpu.VMEM((2, GRP, CHUNK), jnp.int32),
            pltpu.VMEM((CHUNK, D), jnp.float32),
            pltpu.VMEM((CHUNK, D), jnp.float32),
            pltpu.SemaphoreType.DMA,
            pltpu.SemaphoreType.DMA,
            pltpu.SemaphoreType.DMA,
        ],
    )
    def hop(g_hbm, src_hbm, dst_hbm, out_hbm, acc, src_v, dst_v, rows0, rows1,
            sem0, sem1, semi):
        cid = lax.axis_index("c")
        sid = lax.axis_index("s")

        @pl.loop(0, CHUNK)
        def _(r):
            @pl.loop(0, D, step=16)
            def _(c):
                rows0[r, pl.ds(c, 16)] = jnp.zeros((16,), jnp.float32)

        @pl.loop(0, ROWS_PER_SUB, step=CHUNK)
        def _(i):
            pltpu.sync_copy(rows0, acc.at[pl.ds(sid * ROWS_PER_SUB + i, CHUNK)])

        plsc.subcore_barrier()

        def g_start(sv, k, buf, sem):
            pltpu.async_copy(g_hbm.at[sv.at[k]], buf, sem[0])

        def g_wait(sv, k, buf, sem):
            pltpu.make_async_copy(g_hbm.at[sv.at[k]], buf, sem[0]).wait()

        def sc_add(dv, k, buf):
            pltpu.sync_copy(buf, acc.at[dv.at[k]], add=True)

        def run(bg, ngr):
            pltpu.sync_copy(src_hbm.at[bg], src_v.at[0])
            pltpu.sync_copy(dst_hbm.at[bg], dst_v.at[0])
            g_start(src_v.at[0], 0, rows0, (sem0,))

            def group(t, p):
                @pl.when(t + 1 < ngr)
                def _():
                    pltpu.async_copy(src_hbm.at[bg + t + 1], src_v.at[1 - p], semi)
                    pltpu.async_copy(dst_hbm.at[bg + t + 1], dst_v.at[1 - p], semi)

                sv = src_v.at[p]
                dv = dst_v.at[p]
                svn = src_v.at[1 - p]

                @pl.loop(0, GRP, step=2)
                def _(k):
                    g_start(sv, k + 1, rows1, (sem1,))
                    g_wait(sv, k, rows0, (sem0,))
                    sc_add(dv, k, rows0)

                    @pl.when(k + 2 < GRP)
                    def _():
                        g_start(sv, k + 2, rows0, (sem0,))

                    @pl.when((k + 2 >= GRP) & (t + 1 < ngr))
                    def _():
                        pltpu.make_async_copy(src_hbm.at[bg + t + 1],
                                              src_v.at[1 - p], semi).wait()
                        pltpu.make_async_copy(dst_hbm.at[bg + t + 1],
                                              dst_v.at[1 - p], semi).wait()
                        g_start(svn, 0, rows0, (sem0,))

                    g_wait(sv, k + 1, rows1, (sem1,))
                    sc_add(dv, k + 1, rows1)

            @pl.loop(0, ngr, step=2)
            def _(t):
                group(t, 0)
                group(t + 1, 1)

        run((sid * NC + cid) * GW, GW)

        plsc.subcore_barrier()
        pltpu.sync_copy(
            acc.at[pl.ds(sid * ROWS_PER_SUB, ROWS_PER_SUB)],
            out_hbm.at[cid, pl.ds(sid * ROWS_PER_SUB, ROWS_PER_SUB)],
        )

    return hop(g, src_rows, dst_rows)


def _deg_col(deg_ref):
    return jnp.sum(deg_ref[...], axis=1, keepdims=True) + 1.0


def _tc_lin_scale(x_pad, W, degt):
    def body(x_ref, w_ref, deg_ref, g_ref):
        y = lax.dot_general(x_ref[...], w_ref[...], (((1,), (1,)), ((), ())),
                            preferred_element_type=jnp.float32)
        g_ref[...] = y * lax.rsqrt(_deg_col(deg_ref))

    return pl.pallas_call(
        body,
        grid=(N_PAD // TC_BLK,),
        in_specs=[
            pl.BlockSpec((TC_BLK, D), lambda i: (i, 0)),
            pl.BlockSpec((D, D), lambda i: (0, 0)),
            pl.BlockSpec((TC_BLK, NC), lambda i: (i, 0)),
        ],
        out_specs=pl.BlockSpec((TC_BLK, D), lambda i: (i, 0)),
        out_shape=jax.ShapeDtypeStruct((N_PAD, D), jnp.float32),
    )(x_pad, W, degt)


def _tc_mid(accp, g1, degt):
    def body(a_ref, g_ref, deg_ref, o_ref):
        s = a_ref[0] + a_ref[1] + g_ref[...]
        o_ref[...] = s / _deg_col(deg_ref)

    return pl.pallas_call(
        body,
        grid=(N_PAD // TC_BLK,),
        in_specs=[
            pl.BlockSpec((NC, TC_BLK, D), lambda i: (0, i, 0)),
            pl.BlockSpec((TC_BLK, D), lambda i: (i, 0)),
            pl.BlockSpec((TC_BLK, NC), lambda i: (i, 0)),
        ],
        out_specs=pl.BlockSpec((TC_BLK, D), lambda i: (i, 0)),
        out_shape=jax.ShapeDtypeStruct((N_PAD, D), jnp.float32),
    )(accp, g1, degt)


def _tc_final(accp, g2, degt, b2d):
    def body(a_ref, g_ref, deg_ref, b_ref, o_ref):
        s = a_ref[0] + a_ref[1] + g_ref[...]
        o_ref[...] = s * lax.rsqrt(_deg_col(deg_ref)) + b_ref[...]

    return pl.pallas_call(
        body,
        grid=(N_PAD // TC_BLK,),
        in_specs=[
            pl.BlockSpec((NC, TC_BLK, D), lambda i: (0, i, 0)),
            pl.BlockSpec((TC_BLK, D), lambda i: (i, 0)),
            pl.BlockSpec((TC_BLK, NC), lambda i: (i, 0)),
            pl.BlockSpec((1, D), lambda i: (0, 0)),
        ],
        out_specs=pl.BlockSpec((TC_BLK, D), lambda i: (i, 0)),
        out_shape=jax.ShapeDtypeStruct((N_PAD, D), jnp.float32),
    )(accp, g2, degt, b2d)


def kernel(x, edge_index, W, b):
    n = x.shape[0]
    e = edge_index.shape[1]
    src = edge_index[0].astype(jnp.int32)
    dst = edge_index[1].astype(jnp.int32)
    pad = n + (jnp.arange(E_PAD - e, dtype=jnp.int32) % (N_PAD - n))
    src_rows = jnp.concatenate([src, pad]).reshape(NGRP_TOT, GRP, CHUNK)
    dst_rows = jnp.concatenate([dst, pad]).reshape(NGRP_TOT, GRP, CHUNK)
    x_pad = jnp.pad(x, ((0, N_PAD - n), (0, 0)))

    degp = _sc_degree_hist(dst_rows)
    degt = degp.T
    g1 = _tc_lin_scale(x_pad, W, degt)
    acc1 = _sc_hop(g1, src_rows, dst_rows)
    g2 = _tc_mid(acc1, g1, degt)
    acc2 = _sc_hop(g2, src_rows, dst_rows)
    out = _tc_final(acc2, g2, degt, b.reshape(1, D))
    return out[:n]


# speedup vs baseline: 1.1037x; 1.0018x over previous
import functools

import jax
import jax.numpy as jnp
from jax import lax
from jax.experimental import pallas as pl
from jax.experimental.pallas import tpu as pltpu
from jax.experimental.pallas import tpu_sc as plsc

D = 128
NC = 2
NS = 16
NW = NC * NS
N_PAD = 10240
ROWS_PER_SUB = N_PAD // NS
CHUNK = 128
GRP = 8
GW = 10
NGRP_TOT = NW * GW
E_PAD = NGRP_TOT * GRP * CHUNK
TC_BLK = 2048

_sc_mesh = functools.partial(
    plsc.VectorSubcoreMesh, core_axis_name="c", subcore_axis_name="s"
)


def _sc_degree_hist(dst_rows):
    @functools.partial(
        pl.kernel,
        out_type=jax.ShapeDtypeStruct((NC, N_PAD), jnp.float32),
        mesh=_sc_mesh(),
        scratch_types=[
            pltpu.VMEM_SHARED((N_PAD,), jnp.float32),
            pltpu.VMEM((GW, GRP, CHUNK), jnp.int32),
            pltpu.VMEM((ROWS_PER_SUB,), jnp.float32),
            pltpu.VMEM((CHUNK,), jnp.float32),
        ],
    )
    def hist(dst_hbm, out_hbm, acc, dst_v, zero_v, ones_v):
        cid = lax.axis_index("c")
        sid = lax.axis_index("s")

        @pl.loop(0, ROWS_PER_SUB, step=16)
        def _(i):
            zero_v[pl.ds(i, 16)] = jnp.zeros((16,), jnp.float32)

        @pl.loop(0, CHUNK, step=16)
        def _(i):
            ones_v[pl.ds(i, 16)] = jnp.ones((16,), jnp.float32)

        pltpu.sync_copy(zero_v, acc.at[pl.ds(sid * ROWS_PER_SUB, ROWS_PER_SUB)])
        plsc.subcore_barrier()

        wid = sid * NC + cid
        pltpu.sync_copy(dst_hbm.at[pl.ds(wid * GW, GW)], dst_v)

        @pl.loop(0, GW)
        def _(t):
            @pl.loop(0, GRP)
            def _(k):
                pltpu.sync_copy(ones_v, acc.at[dst_v.at[t, k]], add=True)

        plsc.subcore_barrier()
        pltpu.sync_copy(
            acc.at[pl.ds(sid * ROWS_PER_SUB, ROWS_PER_SUB)],
            out_hbm.at[cid, pl.ds(sid * ROWS_PER_SUB, ROWS_PER_SUB)],
        )

    return hist(dst_rows)


def _sc_hop(g, src_rows, dst_rows):
    @functools.partial(
        pl.kernel,
        out_type=jax.ShapeDtypeStruct((NC, N_PAD, D), jnp.float32),
        mesh=_sc_mesh(),
        scratch_types=[
            pltpu.VMEM_SHARED((N_PAD, D), jnp.float32),
            pltpu.VMEM((2, GRP, CHUNK), jnp.int32),
            pltpu.VMEM((2, GRP, CHUNK), jnp.int32),
            pltpu.VMEM((CHUNK, D), jnp.float32),
            pltpu.VMEM((CHUNK, D), jnp.float32),
            pltpu.SemaphoreType.DMA,
            pltpu.SemaphoreType.DMA,
            pltpu.SemaphoreType.DMA,
        ],
    )
    def hop(g_hbm, src_hbm, dst_hbm, out_hbm, acc, src_v, dst_v, rows0, rows1,
            sem0, sem1, semi):
        cid = lax.axis_index("c")
        sid = lax.axis_index("s")

        @pl.loop(0, CHUNK)
        def _(r):
            @pl.loop(0, D, step=16)
            def _(c):
                rows0[r, pl.ds(c, 16)] = jnp.zeros((16,), jnp.float32)

        @pl.loop(0, ROWS_PER_SUB, step=CHUNK)
        def _(i):
            pltpu.sync_copy(rows0, acc.at[pl.ds(sid * ROWS_PER_SUB + i, CHUNK)])

        plsc.subcore_barrier()

        def g_start(sv, k, buf, sem):
            pltpu.async_copy(g_hbm.at[sv.at[k]], buf, sem[0])

        def g_wait(sv, k, buf, sem):
            pltpu.make_async_copy(g_hbm.at[sv.at[k]], buf, sem[0]).wait()

        def sc_add(dv, k, buf):
            pltpu.sync_copy(buf, acc.at[dv.at[k]], add=True)

        def run(bg, ngr):
            pltpu.sync_copy(src_hbm.at[bg], src_v.at[0])
            pltpu.sync_copy(dst_hbm.at[bg], dst_v.at[0])
            g_start(src_v.at[0], 0, rows0, (sem0,))

            def group(t, p):
                @pl.when(t + 1 < ngr)
                def _():
                    pltpu.async_copy(src_hbm.at[bg + t + 1], src_v.at[1 - p], semi)
                    pltpu.async_copy(dst_hbm.at[bg + t + 1], dst_v.at[1 - p], semi)

                sv = src_v.at[p]
                dv = dst_v.at[p]
                svn = src_v.at[1 - p]

                @pl.loop(0, GRP, step=2)
                def _(k):
                    g_start(sv, k + 1, rows1, (sem1,))
                    g_wait(sv, k, rows0, (sem0,))
                    sc_add(dv, k, rows0)

                    @pl.when(k + 2 < GRP)
                    def _():
                        g_start(sv, k + 2, rows0, (sem0,))

                    @pl.when((k + 2 >= GRP) & (t + 1 < ngr))
                    def _():
                        pltpu.make_async_copy(src_hbm.at[bg + t + 1],
                                              src_v.at[1 - p], semi).wait()
                        pltpu.make_async_copy(dst_hbm.at[bg + t + 1],
                                              dst_v.at[1 - p], semi).wait()
                        g_start(svn, 0, rows0, (sem0,))

                    g_wait(sv, k + 1, rows1, (sem1,))
                    sc_add(dv, k + 1, rows1)

            @pl.loop(0, ngr, step=2)
            def _(t):
                group(t, 0)
                group(t + 1, 1)

        run((sid * NC + cid) * GW, GW)

        plsc.subcore_barrier()
        pltpu.sync_copy(
            acc.at[pl.ds(sid * ROWS_PER_SUB, ROWS_PER_SUB)],
            out_hbm.at[cid, pl.ds(sid * ROWS_PER_SUB, ROWS_PER_SUB)],
        )

    return hop(g, src_rows, dst_rows)


def _deg_col(deg_ref):
    return jnp.sum(deg_ref[...], axis=1, keepdims=True) + 1.0


def _tc_matmul(x_pad, W):
    def body(x_ref, w_ref, y_ref):
        y_ref[...] = lax.dot_general(x_ref[...], w_ref[...],
                                     (((1,), (1,)), ((), ())),
                                     preferred_element_type=jnp.float32)

    return pl.pallas_call(
        body,
        grid=(N_PAD // TC_BLK,),
        in_specs=[
            pl.BlockSpec((TC_BLK, D), lambda i: (i, 0)),
            pl.BlockSpec((D, D), lambda i: (0, 0)),
        ],
        out_specs=pl.BlockSpec((TC_BLK, D), lambda i: (i, 0)),
        out_shape=jax.ShapeDtypeStruct((N_PAD, D), jnp.float32),
    )(x_pad, W)


def _tc_scale(y, degt):
    def body(y_ref, deg_ref, g_ref):
        g_ref[...] = y_ref[...] * lax.rsqrt(_deg_col(deg_ref))

    return pl.pallas_call(
        body,
        grid=(N_PAD // TC_BLK,),
        in_specs=[
            pl.BlockSpec((TC_BLK, D), lambda i: (i, 0)),
            pl.BlockSpec((TC_BLK, NC), lambda i: (i, 0)),
        ],
        out_specs=pl.BlockSpec((TC_BLK, D), lambda i: (i, 0)),
        out_shape=jax.ShapeDtypeStruct((N_PAD, D), jnp.float32),
    )(y, degt)


def _tc_mid(accp, g1, degt):
    def body(a_ref, g_ref, deg_ref, o_ref):
        s = a_ref[0] + a_ref[1] + g_ref[...]
        o_ref[...] = s / _deg_col(deg_ref)

    return pl.pallas_call(
        body,
        grid=(N_PAD // TC_BLK,),
        in_specs=[
            pl.BlockSpec((NC, TC_BLK, D), lambda i: (0, i, 0)),
            pl.BlockSpec((TC_BLK, D), lambda i: (i, 0)),
            pl.BlockSpec((TC_BLK, NC), lambda i: (i, 0)),
        ],
        out_specs=pl.BlockSpec((TC_BLK, D), lambda i: (i, 0)),
        out_shape=jax.ShapeDtypeStruct((N_PAD, D), jnp.float32),
    )(accp, g1, degt)


def _tc_final(accp, g2, degt, b2d):
    def body(a_ref, g_ref, deg_ref, b_ref, o_ref):
        s = a_ref[0] + a_ref[1] + g_ref[...]
        o_ref[...] = s * lax.rsqrt(_deg_col(deg_ref)) + b_ref[...]

    return pl.pallas_call(
        body,
        grid=(N_PAD // TC_BLK,),
        in_specs=[
            pl.BlockSpec((NC, TC_BLK, D), lambda i: (0, i, 0)),
            pl.BlockSpec((TC_BLK, D), lambda i: (i, 0)),
            pl.BlockSpec((TC_BLK, NC), lambda i: (i, 0)),
            pl.BlockSpec((1, D), lambda i: (0, 0)),
        ],
        out_specs=pl.BlockSpec((TC_BLK, D), lambda i: (i, 0)),
        out_shape=jax.ShapeDtypeStruct((N_PAD, D), jnp.float32),
    )(accp, g2, degt, b2d)


def kernel(x, edge_index, W, b):
    n = x.shape[0]
    e = edge_index.shape[1]
    src = edge_index[0].astype(jnp.int32)
    dst = edge_index[1].astype(jnp.int32)
    pad = n + (jnp.arange(E_PAD - e, dtype=jnp.int32) % (N_PAD - n))
    src_rows = jnp.concatenate([src, pad]).reshape(NGRP_TOT, GRP, CHUNK)
    dst_rows = jnp.concatenate([dst, pad]).reshape(NGRP_TOT, GRP, CHUNK)
    x_pad = jnp.pad(x, ((0, N_PAD - n), (0, 0)))

    y = _tc_matmul(x_pad, W)
    degp = _sc_degree_hist(dst_rows)
    degt = degp.T
    g1 = _tc_scale(y, degt)
    acc1 = _sc_hop(g1, src_rows, dst_rows)
    g2 = _tc_mid(acc1, g1, degt)
    acc2 = _sc_hop(g2, src_rows, dst_rows)
    out = _tc_final(acc2, g2, degt, b.reshape(1, D))
    return out[:n]


# speedup vs baseline: 1.1297x; 1.0235x over previous
import functools

import jax
import jax.numpy as jnp
from jax import lax
from jax.experimental import pallas as pl
from jax.experimental.pallas import tpu as pltpu
from jax.experimental.pallas import tpu_sc as plsc

D = 128
NC = 2
NS = 16
NW = NC * NS
N_PAD = 10240
ROWS_PER_SUB = N_PAD // NS
CHUNK = 128
GRP = 8
GW = 10
NGRP_TOT = NW * GW
E_PAD = NGRP_TOT * GRP * CHUNK
TC_BLK = 5120

_sc_mesh = functools.partial(
    plsc.VectorSubcoreMesh, core_axis_name="c", subcore_axis_name="s"
)


def _sc_degree_hist(dst_rows):
    @functools.partial(
        pl.kernel,
        out_type=jax.ShapeDtypeStruct((NC, N_PAD), jnp.float32),
        mesh=_sc_mesh(),
        scratch_types=[
            pltpu.VMEM_SHARED((N_PAD,), jnp.float32),
            pltpu.VMEM((GW, GRP, CHUNK), jnp.int32),
            pltpu.VMEM((ROWS_PER_SUB,), jnp.float32),
            pltpu.VMEM((CHUNK,), jnp.float32),
            pltpu.SemaphoreType.DMA,
        ],
    )
    def hist(dst_hbm, out_hbm, acc, dst_v, zero_v, ones_v, sem):
        cid = lax.axis_index("c")
        sid = lax.axis_index("s")

        @pl.loop(0, ROWS_PER_SUB, step=16)
        def _(i):
            zero_v[pl.ds(i, 16)] = jnp.zeros((16,), jnp.float32)

        @pl.loop(0, CHUNK, step=16)
        def _(i):
            ones_v[pl.ds(i, 16)] = jnp.ones((16,), jnp.float32)

        pltpu.sync_copy(zero_v, acc.at[pl.ds(sid * ROWS_PER_SUB, ROWS_PER_SUB)])
        plsc.subcore_barrier()

        wid = sid * NC + cid
        pltpu.sync_copy(dst_hbm.at[pl.ds(wid * GW, GW)], dst_v)

        @pl.loop(0, GW)
        def _(t):
            @pl.loop(0, GRP)
            def _(k):
                pltpu.async_copy(ones_v, acc.at[dst_v.at[t, k]], sem, add=True)

        @pl.loop(0, GW)
        def _(t):
            @pl.loop(0, GRP)
            def _(k):
                pltpu.make_async_copy(ones_v, acc.at[dst_v.at[t, k]],
                                      sem).wait()

        plsc.subcore_barrier()
        pltpu.sync_copy(
            acc.at[pl.ds(sid * ROWS_PER_SUB, ROWS_PER_SUB)],
            out_hbm.at[cid, pl.ds(sid * ROWS_PER_SUB, ROWS_PER_SUB)],
        )

    return hist(dst_rows)


def _sc_hop(g, src_rows, dst_rows):
    @functools.partial(
        pl.kernel,
        out_type=jax.ShapeDtypeStruct((NC, N_PAD, D), jnp.float32),
        mesh=_sc_mesh(),
        scratch_types=[
            pltpu.VMEM_SHARED((N_PAD, D), jnp.float32),
            pltpu.VMEM((2, GRP, CHUNK), jnp.int32),
            pltpu.VMEM((2, GRP, CHUNK), jnp.int32),
            pltpu.VMEM((CHUNK, D), jnp.float32),
            pltpu.VMEM((CHUNK, D), jnp.float32),
            pltpu.SemaphoreType.DMA,
            pltpu.SemaphoreType.DMA,
            pltpu.SemaphoreType.DMA,
        ],
    )
    def hop(g_hbm, src_hbm, dst_hbm, out_hbm, acc, src_v, dst_v, rows0, rows1,
            sem0, sem1, semi):
        cid = lax.axis_index("c")
        sid = lax.axis_index("s")

        @pl.loop(0, CHUNK)
        def _(r):
            @pl.loop(0, D, step=16)
            def _(c):
                rows0[r, pl.ds(c, 16)] = jnp.zeros((16,), jnp.float32)

        @pl.loop(0, ROWS_PER_SUB, step=CHUNK)
        def _(i):
            pltpu.sync_copy(rows0, acc.at[pl.ds(sid * ROWS_PER_SUB + i, CHUNK)])

        plsc.subcore_barrier()

        def g_start(sv, k, buf, sem):
            pltpu.async_copy(g_hbm.at[sv.at[k]], buf, sem[0])

        def g_wait(sv, k, buf, sem):
            pltpu.make_async_copy(g_hbm.at[sv.at[k]], buf, sem[0]).wait()

        def sc_add(dv, k, buf):
            pltpu.sync_copy(buf, acc.at[dv.at[k]], add=True)

        def run(bg, ngr):
            pltpu.sync_copy(src_hbm.at[bg], src_v.at[0])
            pltpu.sync_copy(dst_hbm.at[bg], dst_v.at[0])
            g_start(src_v.at[0], 0, rows0, (sem0,))

            def group(t, p):
                @pl.when(t + 1 < ngr)
                def _():
                    pltpu.async_copy(src_hbm.at[bg + t + 1], src_v.at[1 - p], semi)
                    pltpu.async_copy(dst_hbm.at[bg + t + 1], dst_v.at[1 - p], semi)

                sv = src_v.at[p]
                dv = dst_v.at[p]
                svn = src_v.at[1 - p]

                @pl.loop(0, GRP, step=2)
                def _(k):
                    g_start(sv, k + 1, rows1, (sem1,))
                    g_wait(sv, k, rows0, (sem0,))
                    sc_add(dv, k, rows0)

                    @pl.when(k + 2 < GRP)
                    def _():
                        g_start(sv, k + 2, rows0, (sem0,))

                    @pl.when((k + 2 >= GRP) & (t + 1 < ngr))
                    def _():
                        pltpu.make_async_copy(src_hbm.at[bg + t + 1],
                                              src_v.at[1 - p], semi).wait()
                        pltpu.make_async_copy(dst_hbm.at[bg + t + 1],
                                              dst_v.at[1 - p], semi).wait()
                        g_start(svn, 0, rows0, (sem0,))

                    g_wait(sv, k + 1, rows1, (sem1,))
                    sc_add(dv, k + 1, rows1)

            @pl.loop(0, ngr, step=2)
            def _(t):
                group(t, 0)
                group(t + 1, 1)

        run((sid * NC + cid) * GW, GW)

        plsc.subcore_barrier()
        pltpu.sync_copy(
            acc.at[pl.ds(sid * ROWS_PER_SUB, ROWS_PER_SUB)],
            out_hbm.at[cid, pl.ds(sid * ROWS_PER_SUB, ROWS_PER_SUB)],
        )

    return hop(g, src_rows, dst_rows)


def _deg_col(deg_ref):
    return jnp.sum(deg_ref[...], axis=1, keepdims=True) + 1.0


def _tc_matmul(x_pad, W):
    def body(x_ref, w_ref, y_ref):
        y_ref[...] = lax.dot_general(x_ref[...], w_ref[...],
                                     (((1,), (1,)), ((), ())),
                                     preferred_element_type=jnp.float32)

    return pl.pallas_call(
        body,
        grid=(N_PAD // TC_BLK,),
        in_specs=[
            pl.BlockSpec((TC_BLK, D), lambda i: (i, 0)),
            pl.BlockSpec((D, D), lambda i: (0, 0)),
        ],
        out_specs=pl.BlockSpec((TC_BLK, D), lambda i: (i, 0)),
        out_shape=jax.ShapeDtypeStruct((N_PAD, D), jnp.float32),
    )(x_pad, W)


def _tc_scale(y, degt):
    def body(y_ref, deg_ref, g_ref):
        g_ref[...] = y_ref[...] * lax.rsqrt(_deg_col(deg_ref))

    return pl.pallas_call(
        body,
        grid=(N_PAD // TC_BLK,),
        in_specs=[
            pl.BlockSpec((TC_BLK, D), lambda i: (i, 0)),
            pl.BlockSpec((TC_BLK, NC), lambda i: (i, 0)),
        ],
        out_specs=pl.BlockSpec((TC_BLK, D), lambda i: (i, 0)),
        out_shape=jax.ShapeDtypeStruct((N_PAD, D), jnp.float32),
    )(y, degt)


def _tc_mid(accp, g1, degt):
    def body(a_ref, g_ref, deg_ref, o_ref):
        s = a_ref[0] + a_ref[1] + g_ref[...]
        o_ref[...] = s / _deg_col(deg_ref)

    return pl.pallas_call(
        body,
        grid=(N_PAD // TC_BLK,),
        in_specs=[
            pl.BlockSpec((NC, TC_BLK, D), lambda i: (0, i, 0)),
            pl.BlockSpec((TC_BLK, D), lambda i: (i, 0)),
            pl.BlockSpec((TC_BLK, NC), lambda i: (i, 0)),
        ],
        out_specs=pl.BlockSpec((TC_BLK, D), lambda i: (i, 0)),
        out_shape=jax.ShapeDtypeStruct((N_PAD, D), jnp.float32),
    )(accp, g1, degt)


def _tc_final(accp, g2, degt, b2d):
    def body(a_ref, g_ref, deg_ref, b_ref, o_ref):
        s = a_ref[0] + a_ref[1] + g_ref[...]
        o_ref[...] = s * lax.rsqrt(_deg_col(deg_ref)) + b_ref[...]

    return pl.pallas_call(
        body,
        grid=(N_PAD // TC_BLK,),
        in_specs=[
            pl.BlockSpec((NC, TC_BLK, D), lambda i: (0, i, 0)),
            pl.BlockSpec((TC_BLK, D), lambda i: (i, 0)),
            pl.BlockSpec((TC_BLK, NC), lambda i: (i, 0)),
            pl.BlockSpec((1, D), lambda i: (0, 0)),
        ],
        out_specs=pl.BlockSpec((TC_BLK, D), lambda i: (i, 0)),
        out_shape=jax.ShapeDtypeStruct((N_PAD, D), jnp.float32),
    )(accp, g2, degt, b2d)


def kernel(x, edge_index, W, b):
    n = x.shape[0]
    e = edge_index.shape[1]
    src = edge_index[0].astype(jnp.int32)
    dst = edge_index[1].astype(jnp.int32)
    pad = n + (jnp.arange(E_PAD - e, dtype=jnp.int32) % (N_PAD - n))
    src_rows = jnp.concatenate([src, pad]).reshape(NGRP_TOT, GRP, CHUNK)
    dst_rows = jnp.concatenate([dst, pad]).reshape(NGRP_TOT, GRP, CHUNK)
    x_pad = jnp.pad(x, ((0, N_PAD - n), (0, 0)))

    y = _tc_matmul(x_pad, W)
    degp = _sc_degree_hist(dst_rows)
    degt = degp.T
    g1 = _tc_scale(y, degt)
    acc1 = _sc_hop(g1, src_rows, dst_rows)
    g2 = _tc_mid(acc1, g1, degt)
    acc2 = _sc_hop(g2, src_rows, dst_rows)
    out = _tc_final(acc2, g2, degt, b.reshape(1, D))
    return out[:n]
